# baseline (speedup 1.0000x reference)
import jax
import jax.numpy as jnp
from jax import lax
from jax.experimental import pallas as pl
from jax.experimental.pallas import tpu as pltpu
from jax.experimental.pallas import tpu_sc as plsc

_NC, _NS = 2, 16
_NW = _NC * _NS
_CHUNK = 512


def _embed_body(tokens_hbm, table_hbm, out_hbm, idx_v, rows_v, gsem):
    n = tokens_hbm.shape[0]
    n_per_w = n // _NW
    chunks = n_per_w // _CHUNK
    wid = lax.axis_index("s") * _NC + lax.axis_index("c")
    base = wid * n_per_w

    def body(g, carry):
        start = base + g * _CHUNK
        pltpu.sync_copy(tokens_hbm.at[pl.ds(start, _CHUNK)], idx_v)
        pltpu.async_copy(table_hbm.at[idx_v], rows_v, gsem).wait()
        pltpu.sync_copy(rows_v, out_hbm.at[pl.ds(start, _CHUNK)])
        return carry

    lax.fori_loop(0, chunks, body, 0)


def kernel(input_tokens, table):
    B, H = input_tokens.shape
    V, D = table.shape
    flat = input_tokens.reshape(B * H).astype(jnp.int32)
    assert (B * H) % (_NW * _CHUNK) == 0

    k = pl.kernel(
        _embed_body,
        out_type=jax.ShapeDtypeStruct((B * H, D), table.dtype),
        mesh=plsc.VectorSubcoreMesh(core_axis_name="c", subcore_axis_name="s"),
        scratch_types=[
            pltpu.VMEM((_CHUNK,), jnp.int32),
            pltpu.VMEM((_CHUNK, D), jnp.float32),
            pltpu.SemaphoreType.DMA,
        ],
        compiler_params=pltpu.CompilerParams(use_tc_tiling_on_sc=False),
    )
    out = k(flat, table)
    return out.reshape(B, H, D)


# speedup vs baseline: 1.0750x; 1.0750x over previous
import jax
import jax.numpy as jnp
from jax import lax
from jax.experimental import pallas as pl
from jax.experimental.pallas import tpu as pltpu
from jax.experimental.pallas import tpu_sc as plsc

_NC, _NS = 2, 16
_NW = _NC * _NS
_CHUNK = 512
_NBUF = 2


def _embed_body(tokens_hbm, table_hbm, out_hbm, idx_v, rows_v, gsem, isem, ssem):
    n = tokens_hbm.shape[0]
    n_per_w = n // _NW
    num_chunks = n_per_w // _CHUNK
    wid = lax.axis_index("s") * _NC + lax.axis_index("c")
    base = wid * n_per_w

    def tok_sl(g):
        return tokens_hbm.at[pl.ds(base + g * _CHUNK, _CHUNK)]

    def out_sl(g):
        return out_hbm.at[pl.ds(base + g * _CHUNK, _CHUNK)]

    for b in range(_NBUF):
        pltpu.sync_copy(tok_sl(b), idx_v.at[b])
        pltpu.async_copy(table_hbm.at[idx_v.at[b]], rows_v.at[b], gsem.at[b])

    def slot(g, b):
        pltpu.make_async_copy(table_hbm.at[idx_v.at[b]], rows_v.at[b],
                              gsem.at[b]).wait()
        pltpu.async_copy(tok_sl(g + _NBUF), idx_v.at[b], isem.at[b])
        pltpu.async_copy(rows_v.at[b], out_sl(g), ssem.at[b])
        pltpu.make_async_copy(tok_sl(g + _NBUF), idx_v.at[b], isem.at[b]).wait()
        pltpu.make_async_copy(rows_v.at[b], out_sl(g), ssem.at[b]).wait()
        pltpu.async_copy(table_hbm.at[idx_v.at[b]], rows_v.at[b], gsem.at[b])

    def body(p, carry):
        g0 = p * _NBUF
        for b in range(_NBUF):
            slot(g0 + b, b)
        return carry

    lax.fori_loop(0, num_chunks // _NBUF - 1, body, 0)

    for b in range(_NBUF):
        g = num_chunks - _NBUF + b
        pltpu.make_async_copy(table_hbm.at[idx_v.at[b]], rows_v.at[b],
                              gsem.at[b]).wait()
        pltpu.async_copy(rows_v.at[b], out_sl(g), ssem.at[b])
    for b in range(_NBUF):
        g = num_chunks - _NBUF + b
        pltpu.make_async_copy(rows_v.at[b], out_sl(g), ssem.at[b]).wait()


def kernel(input_tokens, table):
    B, H = input_tokens.shape
    V, D = table.shape
    flat = input_tokens.reshape(B * H).astype(jnp.int32)
    assert (B * H) % (_NW * _CHUNK * _NBUF) == 0

    k = pl.kernel(
        _embed_body,
        out_type=jax.ShapeDtypeStruct((B * H, D), table.dtype),
        mesh=plsc.VectorSubcoreMesh(core_axis_name="c", subcore_axis_name="s"),
        scratch_types=[
            pltpu.VMEM((_NBUF, _CHUNK), jnp.int32),
            pltpu.VMEM((_NBUF, _CHUNK, D), jnp.float32),
            pltpu.SemaphoreType.DMA((_NBUF,)),
            pltpu.SemaphoreType.DMA((_NBUF,)),
            pltpu.SemaphoreType.DMA((_NBUF,)),
        ],
        compiler_params=pltpu.CompilerParams(use_tc_tiling_on_sc=False),
    )
    out = k(flat, table)
    return out.reshape(B, H, D)


# speedup vs baseline: 1.0776x; 1.0024x over previous
import jax
import jax.numpy as jnp
from jax import lax
from jax.experimental import pallas as pl
from jax.experimental.pallas import tpu as pltpu
from jax.experimental.pallas import tpu_sc as plsc

_NC, _NS = 2, 16
_NW = _NC * _NS
_BPC = 4
_NBUF = 2


def _embed_body(tokens_hbm, table_hbm, out_hbm, idx_v, rows_v, gsem, isem, ssem):
    n = tokens_hbm.shape[0]
    H = out_hbm.shape[1]
    chunk = _BPC * H
    n_per_w = n // _NW
    num_chunks = n_per_w // chunk
    wid = lax.axis_index("s") * _NC + lax.axis_index("c")
    base = wid * n_per_w
    batch0 = wid * (n_per_w // H)

    def tok_sl(g):
        return tokens_hbm.at[pl.ds(base + g * chunk, chunk)]

    def start_stores(g, b):
        for j in range(_BPC):
            pltpu.async_copy(rows_v.at[b, pl.ds(j * H, H)],
                             out_hbm.at[batch0 + g * _BPC + j], ssem.at[b])

    def wait_stores(g, b):
        for j in range(_BPC):
            pltpu.make_async_copy(rows_v.at[b, pl.ds(j * H, H)],
                                  out_hbm.at[batch0 + g * _BPC + j],
                                  ssem.at[b]).wait()

    for b in range(_NBUF):
        pltpu.sync_copy(tok_sl(b), idx_v.at[b])
        pltpu.async_copy(table_hbm.at[idx_v.at[b]], rows_v.at[b], gsem.at[b])

    def slot(g, b):
        pltpu.make_async_copy(table_hbm.at[idx_v.at[b]], rows_v.at[b],
                              gsem.at[b]).wait()
        pltpu.async_copy(tok_sl(g + _NBUF), idx_v.at[b], isem.at[b])
        start_stores(g, b)
        pltpu.make_async_copy(tok_sl(g + _NBUF), idx_v.at[b], isem.at[b]).wait()
        wait_stores(g, b)
        pltpu.async_copy(table_hbm.at[idx_v.at[b]], rows_v.at[b], gsem.at[b])

    def body(p, carry):
        g0 = p * _NBUF
        for b in range(_NBUF):
            slot(g0 + b, b)
        return carry

    lax.fori_loop(0, num_chunks // _NBUF - 1, body, 0)

    for b in range(_NBUF):
        g = num_chunks - _NBUF + b
        pltpu.make_async_copy(table_hbm.at[idx_v.at[b]], rows_v.at[b],
                              gsem.at[b]).wait()
        start_stores(g, b)
    for b in range(_NBUF):
        wait_stores(num_chunks - _NBUF + b, b)


def kernel(input_tokens, table):
    B, H = input_tokens.shape
    V, D = table.shape
    flat = input_tokens.reshape(B * H).astype(jnp.int32)
    chunk = _BPC * H
    assert (B * H) % (_NW * chunk * _NBUF) == 0

    k = pl.kernel(
        _embed_body,
        out_type=jax.ShapeDtypeStruct((B, H, D), table.dtype),
        mesh=plsc.VectorSubcoreMesh(core_axis_name="c", subcore_axis_name="s"),
        scratch_types=[
            pltpu.VMEM((_NBUF, chunk), jnp.int32),
            pltpu.VMEM((_NBUF, chunk, D), jnp.float32),
            pltpu.SemaphoreType.DMA((_NBUF,)),
            pltpu.SemaphoreType.DMA((_NBUF,)),
            pltpu.SemaphoreType.DMA((_NBUF,)),
        ],
        compiler_params=pltpu.CompilerParams(use_tc_tiling_on_sc=False),
    )
    return k(flat, table)


# speedup vs baseline: 1.7610x; 1.6343x over previous
import jax
import jax.numpy as jnp
from jax import lax
from jax.experimental import pallas as pl
from jax.experimental.pallas import tpu as pltpu
from jax.experimental.pallas import tpu_sc as plsc

_NC, _NS = 2, 16
_NW = _NC * _NS
_CHUNK = 512
_NBUF = 2


def _embed_body(tokens_hbm, oidx_hbm, table_hbm, out_hbm,
                idx_v, oidx_v, rows_v, gsem, isem, osem, ssem):
    n = tokens_hbm.shape[0]
    n_per_w = n // _NW
    num_chunks = n_per_w // _CHUNK
    wid = lax.axis_index("s") * _NC + lax.axis_index("c")
    base = wid * n_per_w

    def tok_sl(g):
        return tokens_hbm.at[pl.ds(base + g * _CHUNK, _CHUNK)]

    def oix_sl(g):
        return oidx_hbm.at[pl.ds(base + g * _CHUNK, _CHUNK)]

    for b in range(_NBUF):
        pltpu.sync_copy(tok_sl(b), idx_v.at[b])
        pltpu.sync_copy(oix_sl(b), oidx_v.at[b])
        pltpu.async_copy(table_hbm.at[idx_v.at[b]], rows_v.at[b], gsem.at[b])

    def slot(g, b):
        pltpu.make_async_copy(table_hbm.at[idx_v.at[b]], rows_v.at[b],
                              gsem.at[b]).wait()
        pltpu.async_copy(tok_sl(g + _NBUF), idx_v.at[b], isem.at[b])
        pltpu.async_copy(rows_v.at[b], out_hbm.at[oidx_v.at[b]], ssem.at[b])
        pltpu.make_async_copy(tok_sl(g + _NBUF), idx_v.at[b], isem.at[b]).wait()
        pltpu.make_async_copy(rows_v.at[b], out_hbm.at[oidx_v.at[b]],
                              ssem.at[b]).wait()
        pltpu.async_copy(oix_sl(g + _NBUF), oidx_v.at[b], osem.at[b])
        pltpu.make_async_copy(oix_sl(g + _NBUF), oidx_v.at[b], osem.at[b]).wait()
        pltpu.async_copy(table_hbm.at[idx_v.at[b]], rows_v.at[b], gsem.at[b])

    def body(p, carry):
        g0 = p * _NBUF
        for b in range(_NBUF):
            slot(g0 + b, b)
        return carry

    lax.fori_loop(0, num_chunks // _NBUF - 1, body, 0)

    for b in range(_NBUF):
        pltpu.make_async_copy(table_hbm.at[idx_v.at[b]], rows_v.at[b],
                              gsem.at[b]).wait()
        pltpu.async_copy(rows_v.at[b], out_hbm.at[oidx_v.at[b]], ssem.at[b])
    for b in range(_NBUF):
        pltpu.make_async_copy(rows_v.at[b], out_hbm.at[oidx_v.at[b]],
                              ssem.at[b]).wait()


def kernel(input_tokens, table):
    B, H = input_tokens.shape
    V, D = table.shape
    n = B * H
    flat = input_tokens.reshape(n).astype(jnp.int32)
    oidx = jnp.arange(n, dtype=jnp.int32) * 2
    assert n % (_NW * _CHUNK * _NBUF) == 0

    k = pl.kernel(
        _embed_body,
        out_type=jax.ShapeDtypeStruct((2 * n, D), table.dtype),
        mesh=plsc.VectorSubcoreMesh(core_axis_name="c", subcore_axis_name="s"),
        scratch_types=[
            pltpu.VMEM((_NBUF, _CHUNK), jnp.int32),
            pltpu.VMEM((_NBUF, _CHUNK), jnp.int32),
            pltpu.VMEM((_NBUF, _CHUNK, D), jnp.float32),
            pltpu.SemaphoreType.DMA((_NBUF,)),
            pltpu.SemaphoreType.DMA((_NBUF,)),
            pltpu.SemaphoreType.DMA((_NBUF,)),
            pltpu.SemaphoreType.DMA((_NBUF,)),
        ],
        compiler_params=pltpu.CompilerParams(use_tc_tiling_on_sc=False),
    )
    out2 = k(flat, oidx, table)
    out128 = out2.reshape(n, 2 * D)
    return out128[:, :D].reshape(B, H, D)
